# Initial kernel scaffold; baseline (speedup 1.0000x reference)
import jax
import jax.numpy as jnp
from jax.experimental import pallas as pl


def kernel(x, edge_index, batch, W1, b1, W2, b2, Wl, bl):
    raise NotImplementedError("write your pallas kernel here")


# speedup vs baseline: 38.5694x; 38.5694x over previous
import functools

import jax
import jax.numpy as jnp
from jax import lax
from jax.experimental import pallas as pl
from jax.experimental.pallas import tpu as pltpu
from jax.experimental.pallas import tpu_sc as plsc

N = 10000
E = 320000
D = 128
H = 64
G = 64

NC = 2
NS = 16
NW = NC * NS

NA = 10240
CH = 128
K = 80
EPT = K * CH
EPAD = NW * EPT
NBUF = 8
ROWS_PER_SUBCORE = NA // NS

_mesh = lambda: plsc.VectorSubcoreMesh(
    core_axis_name="c", subcore_axis_name="s", num_cores=NC, num_subcores=NS)


def _deg_body(col_hbm, zeros_hbm, out_hbm, idx_v, hist_v):
    cid = lax.axis_index("c")
    sid = lax.axis_index("s")
    t = cid * NS + sid
    pltpu.sync_copy(zeros_hbm, hist_v)
    pltpu.sync_copy(col_hbm.at[t], idx_v)
    ones = jnp.ones((16,), jnp.float32)

    def body(k, _):
        idx = idx_v[pl.ds(k * 16, 16)]
        plsc.addupdate_scatter(hist_v, [idx], ones)
        return 0

    lax.fori_loop(0, EPT // 16, body, 0)
    pltpu.sync_copy(hist_v, out_hbm.at[t])


def _deg_partials(col_tiles, zeros1d):
    return pl.kernel(
        _deg_body,
        out_type=jax.ShapeDtypeStruct((NW, NA), jnp.float32),
        mesh=_mesh(),
        scratch_types=[
            pltpu.VMEM((EPT,), jnp.int32),
            pltpu.VMEM((NA,), jnp.float32),
        ],
        compiler_params=pltpu.CompilerParams(needs_layout_passes=False),
    )(col_tiles, zeros1d)


def _edge_body(y_hbm, row_hbm, col_hbm, zeros_hbm, out_hbm,
               row_v, col_v, data, sems, acc):
    cid = lax.axis_index("c")
    sid = lax.axis_index("s")
    t = cid * NS + sid
    r0 = sid * ROWS_PER_SUBCORE
    pltpu.sync_copy(zeros_hbm.at[pl.ds(r0, ROWS_PER_SUBCORE)],
                    acc.at[pl.ds(r0, ROWS_PER_SUBCORE)])
    pltpu.sync_copy(row_hbm.at[t], row_v)
    pltpu.sync_copy(col_hbm.at[t], col_v)
    plsc.subcore_barrier()

    def super_body(s, _):
        descs = []
        for u in range(NBUF):
            j = s * NBUF + u
            descs.append(
                pltpu.async_copy(y_hbm.at[row_v.at[j]], data[u], sems[u]))
        for u in range(NBUF):
            j = s * NBUF + u
            descs[u].wait()
            pltpu.sync_copy(data[u], acc.at[col_v.at[j]], add=True)
        return 0

    lax.fori_loop(0, K // NBUF, super_body, 0)
    plsc.subcore_barrier()
    pltpu.sync_copy(acc.at[pl.ds(r0, ROWS_PER_SUBCORE)],
                    out_hbm.at[cid, pl.ds(r0, ROWS_PER_SUBCORE)])


def _edge_partials(y, row_tiles, col_tiles, zeros2d):
    def body(y_hbm, row_hbm, col_hbm, zeros_hbm, out_hbm, *scratch):
        row_v, col_v = scratch[0], scratch[1]
        data = scratch[2:2 + NBUF]
        sems = scratch[2 + NBUF:2 + 2 * NBUF]
        acc = scratch[2 + 2 * NBUF]
        _edge_body(y_hbm, row_hbm, col_hbm, zeros_hbm, out_hbm,
                   row_v, col_v, data, sems, acc)

    scratch = [
        pltpu.VMEM((K, CH), jnp.int32),
        pltpu.VMEM((K, CH), jnp.int32),
    ]
    scratch += [pltpu.VMEM((CH, H), jnp.float32) for _ in range(NBUF)]
    scratch += [pltpu.SemaphoreType.DMA for _ in range(NBUF)]
    scratch += [pltpu.VMEM_SHARED((NA, H), jnp.float32)]
    return pl.kernel(
        body,
        out_type=jax.ShapeDtypeStruct((NC, NA, H), jnp.float32),
        mesh=_mesh(),
        scratch_types=scratch,
        compiler_params=pltpu.CompilerParams(use_tc_tiling_on_sc=False),
    )(y, row_tiles, col_tiles, zeros2d)


_BR = 1024
_GRID = NA // _BR


def _tc1_body(hist_ref, x_ref, w_ref, y_ref, dis_ref):
    j = pl.program_id(0)
    deg = jnp.sum(hist_ref[...], axis=0, keepdims=True) + 1.0
    dis = lax.transpose(lax.rsqrt(deg), (1, 0))
    rows = j * _BR + lax.broadcasted_iota(jnp.int32, (_BR, 1), 0)
    valid = rows < N
    dis = jnp.where(valid, dis, 0.0)
    xw = jnp.dot(x_ref[...], w_ref[...], preferred_element_type=jnp.float32)
    y_ref[...] = jnp.where(valid, xw * dis, 0.0)
    dis_ref[...] = dis


def _tc1(hist, x, w1):
    return pl.pallas_call(
        _tc1_body,
        grid=(_GRID,),
        in_specs=[
            pl.BlockSpec((NW, _BR), lambda j: (0, j)),
            pl.BlockSpec((_BR, D), lambda j: (j, 0)),
            pl.BlockSpec((D, H), lambda j: (0, 0)),
        ],
        out_specs=[
            pl.BlockSpec((_BR, H), lambda j: (j, 0)),
            pl.BlockSpec((_BR, 1), lambda j: (j, 0)),
        ],
        out_shape=[
            jax.ShapeDtypeStruct((NA, H), jnp.float32),
            jax.ShapeDtypeStruct((NA, 1), jnp.float32),
        ],
    )(hist, x, w1)


def _tc2_body(p_ref, y_ref, dis_ref, b_ref, w_ref, out_ref):
    p = p_ref[0] + p_ref[1] + y_ref[...]
    h = jnp.maximum(p * dis_ref[...] + b_ref[...], 0.0)
    hw = jnp.dot(h, w_ref[...], preferred_element_type=jnp.float32)
    out_ref[...] = hw * dis_ref[...]


def _tc2(partials, y, dis, b1, w2):
    return pl.pallas_call(
        _tc2_body,
        grid=(_GRID,),
        in_specs=[
            pl.BlockSpec((NC, _BR, H), lambda j: (0, j, 0)),
            pl.BlockSpec((_BR, H), lambda j: (j, 0)),
            pl.BlockSpec((_BR, 1), lambda j: (j, 0)),
            pl.BlockSpec((1, H), lambda j: (0, 0)),
            pl.BlockSpec((H, H), lambda j: (0, 0)),
        ],
        out_specs=pl.BlockSpec((_BR, H), lambda j: (j, 0)),
        out_shape=jax.ShapeDtypeStruct((NA, H), jnp.float32),
    )(partials, y, dis, b1.reshape(1, H), w2)


def _tc3_body(p_ref, y_ref, dis_ref, b_ref, batch_ref, wl_ref, bl_ref,
              out_ref, seg_acc, cnt_acc):
    j = pl.program_id(0)

    @pl.when(j == 0)
    def _():
        seg_acc[...] = jnp.zeros((G, H), jnp.float32)
        cnt_acc[...] = jnp.zeros((G, 1), jnp.float32)

    p = p_ref[0] + p_ref[1] + y_ref[...]
    h = jnp.maximum(p * dis_ref[...] + b_ref[...], 0.0)
    gids = lax.broadcasted_iota(jnp.int32, (_BR, G), 1)
    onehot = (batch_ref[...] == gids).astype(jnp.float32)
    seg_acc[...] += lax.dot_general(
        onehot, h, (((0,), (0,)), ((), ())),
        preferred_element_type=jnp.float32)
    cnt_acc[...] += lax.dot_general(
        onehot, jnp.ones((_BR, 1), jnp.float32), (((0,), (0,)), ((), ())),
        preferred_element_type=jnp.float32)

    @pl.when(j == _GRID - 1)
    def _():
        pooled = seg_acc[...] / jnp.maximum(cnt_acc[...], 1.0)
        z = jnp.dot(pooled, wl_ref[...],
                    preferred_element_type=jnp.float32) + bl_ref[...]
        out_ref[...] = jax.nn.sigmoid(z)


def _tc3(partials, y, dis, b2, batch_pad, wl, bl):
    return pl.pallas_call(
        _tc3_body,
        grid=(_GRID,),
        in_specs=[
            pl.BlockSpec((NC, _BR, H), lambda j: (0, j, 0)),
            pl.BlockSpec((_BR, H), lambda j: (j, 0)),
            pl.BlockSpec((_BR, 1), lambda j: (j, 0)),
            pl.BlockSpec((1, H), lambda j: (0, 0)),
            pl.BlockSpec((_BR, 1), lambda j: (j, 0)),
            pl.BlockSpec((H, 1), lambda j: (0, 0)),
            pl.BlockSpec((1, 1), lambda j: (0, 0)),
        ],
        out_specs=pl.BlockSpec((G, 1), lambda j: (0, 0)),
        out_shape=jax.ShapeDtypeStruct((G, 1), jnp.float32),
        scratch_shapes=[
            pltpu.VMEM((G, H), jnp.float32),
            pltpu.VMEM((G, 1), jnp.float32),
        ],
    )(partials, y, dis, b2.reshape(1, H), batch_pad, wl, bl.reshape(1, 1))


def kernel(x, edge_index, batch, W1, b1, W2, b2, Wl, bl):
    row = edge_index[0]
    col = edge_index[1]
    npad = EPAD - E
    pad_ids = (N + jnp.arange(npad, dtype=jnp.int32) % (NA - N))
    row_tiles = jnp.concatenate([row, pad_ids]).reshape(NW, K, CH)
    col_tiles = jnp.concatenate([col, pad_ids]).reshape(NW, K, CH)
    col_flat = col_tiles.reshape(NW, EPT)
    zeros1d = jnp.zeros((NA,), jnp.float32)
    zeros2d = jnp.zeros((NA, H), jnp.float32)
    batch_pad = jnp.concatenate(
        [batch, jnp.full((NA - N,), G, jnp.int32)]).reshape(NA, 1)

    hist = _deg_partials(col_flat, zeros1d)
    y1, dis = _tc1(hist, x, W1)
    p1 = _edge_partials(y1, row_tiles, col_tiles, zeros2d)
    y2 = _tc2(p1, y1, dis, b1, W2)
    p2 = _edge_partials(y2, row_tiles, col_tiles, zeros2d)
    return _tc3(p2, y2, dis, b2, batch_pad, Wl, bl)
